# baseline (speedup 1.0000x reference)
import functools
import jax
import jax.numpy as jnp
from jax import lax
from jax.experimental import pallas as pl
from jax.experimental.pallas import tpu as pltpu
from jax.experimental.pallas import tpu_sc as plsc

N_NODES = 10000
D = 256
N_EDGES = 160000
HALF = 5000
ACC_ROWS = 5008
NC, NS = 2, 16
DEG_W = 16
EB = 80
EPT = N_EDGES // NS
EPW = N_EDGES // (NC * NS)


def _mesh():
    return plsc.VectorSubcoreMesh(core_axis_name="c", subcore_axis_name="s")


def _fill2d(ref, rows, val):
    def row(r, _):
        def chunk(j, _):
            ref[r, pl.ds(j * 16, 16)] = jnp.full((16,), val, ref.dtype)
            return 0
        return lax.fori_loop(0, ref.shape[1] // 16, chunk, 0)
    lax.fori_loop(0, rows, row, 0)


def _local_dst(didx_v, lidx_v, lo):
    def chunk(j, _):
        d = didx_v[pl.ds(j * 16, 16)]
        ok = (d >= lo) & (d < lo + HALF)
        lidx_v[pl.ds(j * 16, 16)] = jnp.where(ok, d - lo, HALF)
        return 0
    lax.fori_loop(0, EB // 16, chunk, 0)


def _zero_shared(acc_sh, zero_v, sid):
    n_chunks = ACC_ROWS // 16
    per_tile = (n_chunks + NS - 1) // NS

    def body(i, _):
        c = sid * per_tile + i

        @pl.when(c < n_chunks)
        def _():
            pltpu.sync_copy(zero_v, acc_sh.at[pl.ds(c * 16, 16)])
        return 0
    lax.fori_loop(0, per_tile, body, 0)


def _copy_out(acc_sh, out_hbm, cid, sid, width):
    n_chunks = HALF // 8
    per_tile = (n_chunks + NS - 1) // NS

    def body(i, _):
        c = sid * per_tile + i

        @pl.when(c < n_chunks)
        def _():
            pltpu.sync_copy(acc_sh.at[pl.ds(c * 8, 8)],
                            out_hbm.at[cid, pl.ds(c * 8, 8)])
        return 0
    lax.fori_loop(0, per_tile, body, 0)


def _make_tc_scatter(width, eb):
    def body(dst_ref, msg_ref, acc_ref):
        @pl.when(pl.program_id(0) == 0)
        def _():
            acc_ref[...] = jnp.zeros(acc_ref.shape, acc_ref.dtype)

        def one(e, _):
            d = dst_ref[e]
            acc_ref[pl.ds(d, 1), :] = (acc_ref[pl.ds(d, 1), :]
                                       + msg_ref[pl.ds(e, 1), :])
            return 0
        lax.fori_loop(0, eb, one, 0)

    def scatter(dst, msg):
        return pl.pallas_call(
            body,
            grid=(N_EDGES // eb,),
            in_specs=[pl.BlockSpec((eb,), lambda i: (i,),
                                   memory_space=pltpu.SMEM),
                      pl.BlockSpec((eb, width), lambda i: (i, 0))],
            out_specs=pl.BlockSpec((N_NODES, width), lambda i: (0, 0)),
            out_shape=jax.ShapeDtypeStruct((N_NODES, width), jnp.float32),
        )(dst, msg)

    return scatter


_deg_scatter = _make_tc_scatter(DEG_W, 128)
_row_scatter = _make_tc_scatter(D, 128)


def _make_gather(n_rows, eb):
    per_tile = n_rows // (NC * NS)

    @functools.partial(
        pl.kernel, mesh=_mesh(),
        out_type=jax.ShapeDtypeStruct((n_rows, D), jnp.float32),
        scratch_types=[
            pltpu.VMEM((eb, D), jnp.float32),
            pltpu.VMEM((eb,), jnp.int32),
            pltpu.SemaphoreType.DMA,
        ],
    )
    def gather(tab_hbm, idx_hbm, out_hbm, rows_v, idx_v, sem):
        wid = lax.axis_index("s") * NC + lax.axis_index("c")

        def batch(i, _):
            base = wid * per_tile + i * eb
            pltpu.sync_copy(idx_hbm.at[pl.ds(base, eb)], idx_v)
            pltpu.async_copy(tab_hbm.at[idx_v], rows_v, sem).wait()
            pltpu.sync_copy(rows_v, out_hbm.at[pl.ds(base, eb)])
            return 0
        lax.fori_loop(0, per_tile // eb, batch, 0)

    return gather


_msg_gather = _make_gather(N_EDGES, 40)
_edge_gather = _make_gather(2 * N_EDGES, 80)


_RB = 1000


def _scale0_body(deg_ref, x_ref, g_ref):
    dinv = lax.rsqrt(jnp.maximum(deg_ref[:, 0:1], 1.0))
    g_ref[...] = x_ref[...] * dinv


def _tc_scale0(deg16, x):
    return pl.pallas_call(
        _scale0_body,
        grid=(N_NODES // _RB,),
        in_specs=[pl.BlockSpec((_RB, DEG_W), lambda i: (i, 0)),
                  pl.BlockSpec((_RB, D), lambda i: (i, 0))],
        out_specs=pl.BlockSpec((_RB, D), lambda i: (i, 0)),
        out_shape=jax.ShapeDtypeStruct((N_NODES, D), jnp.float32),
    )(deg16, x)


def _layer_body(s_ref, deg_ref, acc_ref, acco_ref, g_ref):
    dinv = lax.rsqrt(jnp.maximum(deg_ref[:, 0:1], 1.0))
    h = s_ref[...] * dinv
    acco_ref[...] = acc_ref[...] + h
    g_ref[...] = h * dinv


def _tc_layer(s, deg16, acc):
    return pl.pallas_call(
        _layer_body,
        grid=(N_NODES // _RB,),
        in_specs=[pl.BlockSpec((_RB, D), lambda i: (i, 0)),
                  pl.BlockSpec((_RB, DEG_W), lambda i: (i, 0)),
                  pl.BlockSpec((_RB, D), lambda i: (i, 0))],
        out_specs=[pl.BlockSpec((_RB, D), lambda i: (i, 0)),
                   pl.BlockSpec((_RB, D), lambda i: (i, 0))],
        out_shape=[jax.ShapeDtypeStruct((N_NODES, D), jnp.float32),
                   jax.ShapeDtypeStruct((N_NODES, D), jnp.float32)],
    )(s, deg16, acc)


def _proj_body(acc_ref, w1_ref, b1_ref, c_ref):
    j = pl.program_id(0)
    o = acc_ref[...] * 0.25
    b = jnp.where(j == 0, b1_ref[...], jnp.zeros_like(b1_ref[...]))
    c_ref[...] = jnp.dot(o, w1_ref[...],
                         preferred_element_type=jnp.float32) + b


def _tc_proj(acc, w1, b1):
    return pl.pallas_call(
        _proj_body,
        grid=(2, N_NODES // _RB),
        in_specs=[pl.BlockSpec((_RB, D), lambda j, i: (i, 0)),
                  pl.BlockSpec((D, D), lambda j, i: (j, 0)),
                  pl.BlockSpec((1, D), lambda j, i: (0, 0))],
        out_specs=pl.BlockSpec((_RB, D), lambda j, i: (j * (N_NODES // _RB) + i, 0)),
        out_shape=jax.ShapeDtypeStruct((2 * N_NODES, D), jnp.float32),
    )(acc, w1, b1)


_EBLK = 2000


def _mlp_body(ga_ref, gb_ref, w2_ref, b2_ref, o_ref):
    z = jnp.maximum(ga_ref[...] + gb_ref[...], 0.0)
    s = jnp.sum(z * w2_ref[...], axis=1, keepdims=True) + b2_ref[0, 0]
    o_ref[...] = jax.nn.sigmoid(s)


def _tc_mlp(gath, w2t, b2):
    nb = N_EDGES // _EBLK
    return pl.pallas_call(
        _mlp_body,
        grid=(nb,),
        in_specs=[pl.BlockSpec((_EBLK, D), lambda i: (i, 0)),
                  pl.BlockSpec((_EBLK, D), lambda i: (nb + i, 0)),
                  pl.BlockSpec((1, D), lambda i: (0, 0)),
                  pl.BlockSpec((1, 1), lambda i: (0, 0))],
        out_specs=pl.BlockSpec((_EBLK, 1), lambda i: (i, 0)),
        out_shape=jax.ShapeDtypeStruct((N_EDGES, 1), jnp.float32),
    )(gath, gath, w2t, b2)


def kernel(batch_x, batch_edge_index, W1, b1, W2, b2):
    src = batch_edge_index[0]
    dst = batch_edge_index[1]

    ones = jnp.ones((N_EDGES, DEG_W), jnp.float32)
    deg16 = _deg_scatter(dst, ones)
    g = _tc_scale0(deg16, batch_x)
    acc = batch_x
    for _ in range(3):
        msg = _msg_gather(g, src)
        s = _row_scatter(dst, msg)
        acc, g = _tc_layer(s, deg16, acc)

    c_tab = _tc_proj(acc, W1, b1.reshape(1, D))
    idx2 = jnp.concatenate([src, dst + N_NODES])
    gath = _edge_gather(c_tab, idx2)
    return _tc_mlp(gath, W2.reshape(1, D), b2.reshape(1, 1))


# speedup vs baseline: 1.1829x; 1.1829x over previous
import functools
import jax
import jax.numpy as jnp
from jax import lax
from jax.experimental import pallas as pl
from jax.experimental.pallas import tpu as pltpu
from jax.experimental.pallas import tpu_sc as plsc

N_NODES = 10000
D = 256
N_EDGES = 160000
HALF = 5000
ACC_ROWS = 5008
NC, NS = 2, 16
DEG_W = 16
EB = 80
EPT = N_EDGES // NS
EPW = N_EDGES // (NC * NS)


def _mesh():
    return plsc.VectorSubcoreMesh(core_axis_name="c", subcore_axis_name="s")


def _fill2d(ref, rows, val):
    def row(r, _):
        def chunk(j, _):
            ref[r, pl.ds(j * 16, 16)] = jnp.full((16,), val, ref.dtype)
            return 0
        return lax.fori_loop(0, ref.shape[1] // 16, chunk, 0)
    lax.fori_loop(0, rows, row, 0)


def _local_dst(didx_v, lidx_v, lo):
    def chunk(j, _):
        d = didx_v[pl.ds(j * 16, 16)]
        ok = (d >= lo) & (d < lo + HALF)
        lidx_v[pl.ds(j * 16, 16)] = jnp.where(ok, d - lo, HALF)
        return 0
    lax.fori_loop(0, EB // 16, chunk, 0)


def _zero_shared(acc_sh, zero_v, sid):
    n_chunks = ACC_ROWS // 16
    per_tile = (n_chunks + NS - 1) // NS

    def body(i, _):
        c = sid * per_tile + i

        @pl.when(c < n_chunks)
        def _():
            pltpu.sync_copy(zero_v, acc_sh.at[pl.ds(c * 16, 16)])
        return 0
    lax.fori_loop(0, per_tile, body, 0)


def _copy_out(acc_sh, out_hbm, cid, sid, width):
    n_chunks = HALF // 8
    per_tile = (n_chunks + NS - 1) // NS

    def body(i, _):
        c = sid * per_tile + i

        @pl.when(c < n_chunks)
        def _():
            pltpu.sync_copy(acc_sh.at[pl.ds(c * 8, 8)],
                            out_hbm.at[cid, pl.ds(c * 8, 8)])
        return 0
    lax.fori_loop(0, per_tile, body, 0)


NSH = 4


def _make_tc_scatter(width, eb):
    def body(dst_ref, msg_ref, acc_ref):
        @pl.when(pl.program_id(0) == 0)
        def _():
            acc_ref[...] = jnp.zeros(acc_ref.shape, acc_ref.dtype)

        def group(g, _):
            for j in range(NSH):
                e = g * NSH + j
                d = dst_ref[e] + j * N_NODES
                acc_ref[pl.ds(d, 1), :] = (acc_ref[pl.ds(d, 1), :]
                                           + msg_ref[pl.ds(e, 1), :])
            return 0
        lax.fori_loop(0, eb // NSH, group, 0)

    def scatter(dst, msg):
        return pl.pallas_call(
            body,
            grid=(N_EDGES // eb,),
            in_specs=[pl.BlockSpec((eb,), lambda i: (i,),
                                   memory_space=pltpu.SMEM),
                      pl.BlockSpec((eb, width), lambda i: (i, 0))],
            out_specs=pl.BlockSpec((NSH * N_NODES, width), lambda i: (0, 0)),
            out_shape=jax.ShapeDtypeStruct((NSH * N_NODES, width),
                                           jnp.float32),
        )(dst, msg)

    return scatter


_deg_scatter = _make_tc_scatter(DEG_W, 128)
_row_scatter = _make_tc_scatter(D, 128)


def _shadow_specs(width):
    def mk(j):
        return pl.BlockSpec((_RB, width),
                            lambda i, j=j: (j * (N_NODES // _RB) + i, 0))
    return [mk(j) for j in range(NSH)]


def _make_gather(n_rows, eb):
    per_tile = n_rows // (NC * NS)

    @functools.partial(
        pl.kernel, mesh=_mesh(),
        out_type=jax.ShapeDtypeStruct((n_rows, D), jnp.float32),
        scratch_types=[
            pltpu.VMEM((eb, D), jnp.float32),
            pltpu.VMEM((eb,), jnp.int32),
            pltpu.SemaphoreType.DMA,
        ],
    )
    def gather(tab_hbm, idx_hbm, out_hbm, rows_v, idx_v, sem):
        wid = lax.axis_index("s") * NC + lax.axis_index("c")

        def batch(i, _):
            base = wid * per_tile + i * eb
            pltpu.sync_copy(idx_hbm.at[pl.ds(base, eb)], idx_v)
            pltpu.async_copy(tab_hbm.at[idx_v], rows_v, sem).wait()
            pltpu.sync_copy(rows_v, out_hbm.at[pl.ds(base, eb)])
            return 0
        lax.fori_loop(0, per_tile // eb, batch, 0)

    return gather


_msg_gather = _make_gather(N_EDGES, 40)
_edge_gather = _make_gather(2 * N_EDGES, 80)


_RB = 1000


def _scale0_body(d0, d1, d2, d3, x_ref, g_ref, deg_ref):
    deg = d0[...] + d1[...] + d2[...] + d3[...]
    deg_ref[...] = deg
    dinv = lax.rsqrt(jnp.maximum(deg[:, 0:1], 1.0))
    g_ref[...] = x_ref[...] * dinv


def _tc_scale0(deg_sh, x):
    return pl.pallas_call(
        _scale0_body,
        grid=(N_NODES // _RB,),
        in_specs=_shadow_specs(DEG_W) + [pl.BlockSpec((_RB, D),
                                                      lambda i: (i, 0))],
        out_specs=[pl.BlockSpec((_RB, D), lambda i: (i, 0)),
                   pl.BlockSpec((_RB, DEG_W), lambda i: (i, 0))],
        out_shape=[jax.ShapeDtypeStruct((N_NODES, D), jnp.float32),
                   jax.ShapeDtypeStruct((N_NODES, DEG_W), jnp.float32)],
    )(deg_sh, deg_sh, deg_sh, deg_sh, x)


def _layer_body(s0, s1, s2, s3, deg_ref, acc_ref, acco_ref, g_ref):
    dinv = lax.rsqrt(jnp.maximum(deg_ref[:, 0:1], 1.0))
    h = (s0[...] + s1[...] + s2[...] + s3[...]) * dinv
    acco_ref[...] = acc_ref[...] + h
    g_ref[...] = h * dinv


def _tc_layer(s_sh, deg16, acc):
    return pl.pallas_call(
        _layer_body,
        grid=(N_NODES // _RB,),
        in_specs=_shadow_specs(D) + [
            pl.BlockSpec((_RB, DEG_W), lambda i: (i, 0)),
            pl.BlockSpec((_RB, D), lambda i: (i, 0))],
        out_specs=[pl.BlockSpec((_RB, D), lambda i: (i, 0)),
                   pl.BlockSpec((_RB, D), lambda i: (i, 0))],
        out_shape=[jax.ShapeDtypeStruct((N_NODES, D), jnp.float32),
                   jax.ShapeDtypeStruct((N_NODES, D), jnp.float32)],
    )(s_sh, s_sh, s_sh, s_sh, deg16, acc)


def _proj_body(acc_ref, w1_ref, b1_ref, c_ref):
    j = pl.program_id(0)
    o = acc_ref[...] * 0.25
    b = jnp.where(j == 0, b1_ref[...], jnp.zeros_like(b1_ref[...]))
    c_ref[...] = jnp.dot(o, w1_ref[...],
                         preferred_element_type=jnp.float32) + b


def _tc_proj(acc, w1, b1):
    return pl.pallas_call(
        _proj_body,
        grid=(2, N_NODES // _RB),
        in_specs=[pl.BlockSpec((_RB, D), lambda j, i: (i, 0)),
                  pl.BlockSpec((D, D), lambda j, i: (j, 0)),
                  pl.BlockSpec((1, D), lambda j, i: (0, 0))],
        out_specs=pl.BlockSpec((_RB, D), lambda j, i: (j * (N_NODES // _RB) + i, 0)),
        out_shape=jax.ShapeDtypeStruct((2 * N_NODES, D), jnp.float32),
    )(acc, w1, b1)


_EBLK = 2000


def _mlp_body(ga_ref, gb_ref, w2_ref, b2_ref, o_ref):
    z = jnp.maximum(ga_ref[...] + gb_ref[...], 0.0)
    s = jnp.sum(z * w2_ref[...], axis=1, keepdims=True) + b2_ref[0, 0]
    o_ref[...] = jax.nn.sigmoid(s)


def _tc_mlp(gath, w2t, b2):
    nb = N_EDGES // _EBLK
    return pl.pallas_call(
        _mlp_body,
        grid=(nb,),
        in_specs=[pl.BlockSpec((_EBLK, D), lambda i: (i, 0)),
                  pl.BlockSpec((_EBLK, D), lambda i: (nb + i, 0)),
                  pl.BlockSpec((1, D), lambda i: (0, 0)),
                  pl.BlockSpec((1, 1), lambda i: (0, 0))],
        out_specs=pl.BlockSpec((_EBLK, 1), lambda i: (i, 0)),
        out_shape=jax.ShapeDtypeStruct((N_EDGES, 1), jnp.float32),
    )(gath, gath, w2t, b2)


def kernel(batch_x, batch_edge_index, W1, b1, W2, b2):
    src = batch_edge_index[0]
    dst = batch_edge_index[1]

    ones = jnp.ones((N_EDGES, DEG_W), jnp.float32)
    deg_sh = _deg_scatter(dst, ones)
    g, deg16 = _tc_scale0(deg_sh, batch_x)
    acc = batch_x
    for _ in range(3):
        msg = _msg_gather(g, src)
        s_sh = _row_scatter(dst, msg)
        acc, g = _tc_layer(s_sh, deg16, acc)

    c_tab = _tc_proj(acc, W1, b1.reshape(1, D))
    idx2 = jnp.concatenate([src, dst + N_NODES])
    gath = _edge_gather(c_tab, idx2)
    return _tc_mlp(gath, W2.reshape(1, D), b2.reshape(1, 1))


# speedup vs baseline: 1.3523x; 1.1432x over previous
import functools
import jax
import jax.numpy as jnp
from jax import lax
from jax.experimental import pallas as pl
from jax.experimental.pallas import tpu as pltpu
from jax.experimental.pallas import tpu_sc as plsc

N_NODES = 10000
D = 256
N_EDGES = 160000
HALF = 5000
ACC_ROWS = 5008
NC, NS = 2, 16
DEG_W = 16
EB = 80
EPT = N_EDGES // NS
EPW = N_EDGES // (NC * NS)


def _mesh():
    return plsc.VectorSubcoreMesh(core_axis_name="c", subcore_axis_name="s")


def _fill2d(ref, rows, val):
    def row(r, _):
        def chunk(j, _):
            ref[r, pl.ds(j * 16, 16)] = jnp.full((16,), val, ref.dtype)
            return 0
        return lax.fori_loop(0, ref.shape[1] // 16, chunk, 0)
    lax.fori_loop(0, rows, row, 0)


def _local_dst(didx_v, lidx_v, lo):
    def chunk(j, _):
        d = didx_v[pl.ds(j * 16, 16)]
        ok = (d >= lo) & (d < lo + HALF)
        lidx_v[pl.ds(j * 16, 16)] = jnp.where(ok, d - lo, HALF)
        return 0
    lax.fori_loop(0, EB // 16, chunk, 0)


def _zero_shared(acc_sh, zero_v, sid):
    n_chunks = ACC_ROWS // 16
    per_tile = (n_chunks + NS - 1) // NS

    def body(i, _):
        c = sid * per_tile + i

        @pl.when(c < n_chunks)
        def _():
            pltpu.sync_copy(zero_v, acc_sh.at[pl.ds(c * 16, 16)])
        return 0
    lax.fori_loop(0, per_tile, body, 0)


def _copy_out(acc_sh, out_hbm, cid, sid, width):
    n_chunks = HALF // 8
    per_tile = (n_chunks + NS - 1) // NS

    def body(i, _):
        c = sid * per_tile + i

        @pl.when(c < n_chunks)
        def _():
            pltpu.sync_copy(acc_sh.at[pl.ds(c * 8, 8)],
                            out_hbm.at[cid, pl.ds(c * 8, 8)])
        return 0
    lax.fori_loop(0, per_tile, body, 0)


NSH = 4


def _make_tc_scatter(width, eb):
    def body(dst_ref, msg_ref, acc_ref):
        @pl.when(pl.program_id(0) == 0)
        def _():
            acc_ref[...] = jnp.zeros(acc_ref.shape, acc_ref.dtype)

        def group(g, _):
            for j in range(NSH):
                e = g * NSH + j
                d = dst_ref[e] + j * N_NODES
                acc_ref[pl.ds(d, 1), :] = (acc_ref[pl.ds(d, 1), :]
                                           + msg_ref[pl.ds(e, 1), :])
            return 0
        lax.fori_loop(0, eb // NSH, group, 0)

    def scatter(dst, msg):
        return pl.pallas_call(
            body,
            grid=(N_EDGES // eb,),
            in_specs=[pl.BlockSpec((eb,), lambda i: (i,),
                                   memory_space=pltpu.SMEM),
                      pl.BlockSpec((eb, width), lambda i: (i, 0))],
            out_specs=pl.BlockSpec((NSH * N_NODES, width), lambda i: (0, 0)),
            out_shape=jax.ShapeDtypeStruct((NSH * N_NODES, width),
                                           jnp.float32),
        )(dst, msg)

    return scatter


_deg_scatter = _make_tc_scatter(DEG_W, 256)
_row_scatter = _make_tc_scatter(D, 256)


def _shadow_specs(width):
    def mk(j):
        return pl.BlockSpec((_RB, width),
                            lambda i, j=j: (j * (N_NODES // _RB) + i, 0))
    return [mk(j) for j in range(NSH)]


def _make_gather(n_rows, eb):
    per_tile = n_rows // (NC * NS)

    @functools.partial(
        pl.kernel, mesh=_mesh(),
        out_type=jax.ShapeDtypeStruct((n_rows, D), jnp.float32),
        scratch_types=[
            pltpu.VMEM((eb, D), jnp.float32),
            pltpu.VMEM((eb,), jnp.int32),
            pltpu.SemaphoreType.DMA,
        ],
    )
    def gather(tab_hbm, idx_hbm, out_hbm, rows_v, idx_v, sem):
        wid = lax.axis_index("s") * NC + lax.axis_index("c")

        def batch(i, _):
            base = wid * per_tile + i * eb
            pltpu.sync_copy(idx_hbm.at[pl.ds(base, eb)], idx_v)
            pltpu.async_copy(tab_hbm.at[idx_v], rows_v, sem).wait()
            pltpu.sync_copy(rows_v, out_hbm.at[pl.ds(base, eb)])
            return 0
        lax.fori_loop(0, per_tile // eb, batch, 0)

    return gather


_msg_gather = _make_gather(N_EDGES, 40)
_edge_gather = _make_gather(2 * N_EDGES, 80)


_RB = 1000


def _scale0_body(d0, d1, d2, d3, x_ref, g_ref, deg_ref):
    deg = d0[...] + d1[...] + d2[...] + d3[...]
    deg_ref[...] = deg
    dinv = lax.rsqrt(jnp.maximum(deg[:, 0:1], 1.0))
    g_ref[...] = x_ref[...] * dinv


def _tc_scale0(deg_sh, x):
    return pl.pallas_call(
        _scale0_body,
        grid=(N_NODES // _RB,),
        in_specs=_shadow_specs(DEG_W) + [pl.BlockSpec((_RB, D),
                                                      lambda i: (i, 0))],
        out_specs=[pl.BlockSpec((_RB, D), lambda i: (i, 0)),
                   pl.BlockSpec((_RB, DEG_W), lambda i: (i, 0))],
        out_shape=[jax.ShapeDtypeStruct((N_NODES, D), jnp.float32),
                   jax.ShapeDtypeStruct((N_NODES, DEG_W), jnp.float32)],
    )(deg_sh, deg_sh, deg_sh, deg_sh, x)


def _layer_body(s0, s1, s2, s3, deg_ref, acc_ref, acco_ref, g_ref):
    dinv = lax.rsqrt(jnp.maximum(deg_ref[:, 0:1], 1.0))
    h = (s0[...] + s1[...] + s2[...] + s3[...]) * dinv
    acco_ref[...] = acc_ref[...] + h
    g_ref[...] = h * dinv


def _tc_layer(s_sh, deg16, acc):
    return pl.pallas_call(
        _layer_body,
        grid=(N_NODES // _RB,),
        in_specs=_shadow_specs(D) + [
            pl.BlockSpec((_RB, DEG_W), lambda i: (i, 0)),
            pl.BlockSpec((_RB, D), lambda i: (i, 0))],
        out_specs=[pl.BlockSpec((_RB, D), lambda i: (i, 0)),
                   pl.BlockSpec((_RB, D), lambda i: (i, 0))],
        out_shape=[jax.ShapeDtypeStruct((N_NODES, D), jnp.float32),
                   jax.ShapeDtypeStruct((N_NODES, D), jnp.float32)],
    )(s_sh, s_sh, s_sh, s_sh, deg16, acc)


def _proj_body(acc_ref, w1_ref, b1_ref, c_ref):
    j = pl.program_id(0)
    o = acc_ref[...] * 0.25
    b = jnp.where(j == 0, b1_ref[...], jnp.zeros_like(b1_ref[...]))
    c_ref[...] = jnp.dot(o, w1_ref[...],
                         preferred_element_type=jnp.float32) + b


def _tc_proj(acc, w1, b1):
    return pl.pallas_call(
        _proj_body,
        grid=(2, N_NODES // _RB),
        in_specs=[pl.BlockSpec((_RB, D), lambda j, i: (i, 0)),
                  pl.BlockSpec((D, D), lambda j, i: (j, 0)),
                  pl.BlockSpec((1, D), lambda j, i: (0, 0))],
        out_specs=pl.BlockSpec((_RB, D), lambda j, i: (j * (N_NODES // _RB) + i, 0)),
        out_shape=jax.ShapeDtypeStruct((2 * N_NODES, D), jnp.float32),
    )(acc, w1, b1)


_EBLK = 2000


def _mlp_body(ga_ref, gb_ref, w2_ref, b2_ref, o_ref):
    z = jnp.maximum(ga_ref[...] + gb_ref[...], 0.0)
    s = jnp.sum(z * w2_ref[...], axis=1, keepdims=True) + b2_ref[0, 0]
    o_ref[...] = jax.nn.sigmoid(s)


def _tc_mlp(gath, w2t, b2):
    nb = N_EDGES // _EBLK
    return pl.pallas_call(
        _mlp_body,
        grid=(nb,),
        in_specs=[pl.BlockSpec((_EBLK, D), lambda i: (i, 0)),
                  pl.BlockSpec((_EBLK, D), lambda i: (nb + i, 0)),
                  pl.BlockSpec((1, D), lambda i: (0, 0)),
                  pl.BlockSpec((1, 1), lambda i: (0, 0))],
        out_specs=pl.BlockSpec((_EBLK, 1), lambda i: (i, 0)),
        out_shape=jax.ShapeDtypeStruct((N_EDGES, 1), jnp.float32),
    )(gath, gath, w2t, b2)


def kernel(batch_x, batch_edge_index, W1, b1, W2, b2):
    src = batch_edge_index[0]
    dst = batch_edge_index[1]

    ones = jnp.ones((N_EDGES, DEG_W), jnp.float32)
    deg_sh = _deg_scatter(dst, ones)
    g, deg16 = _tc_scale0(deg_sh, batch_x)
    acc = batch_x
    for _ in range(3):
        msg = _msg_gather(g, src)
        s_sh = _row_scatter(dst, msg)
        acc, g = _tc_layer(s_sh, deg16, acc)

    c_tab = _tc_proj(acc, W1, b1.reshape(1, D))
    idx2 = jnp.concatenate([src, dst + N_NODES])
    gath = _edge_gather(c_tab, idx2)
    return _tc_mlp(gath, W2.reshape(1, D), b2.reshape(1, 1))
